# Initial kernel scaffold; baseline (speedup 1.0000x reference)
import jax
import jax.numpy as jnp
from jax.experimental import pallas as pl


def kernel(h, x, edge_index, W1, b1, W2, b2, Wc1, bc1, Wc2, bc2):
    raise NotImplementedError("write your pallas kernel here")


# speedup vs baseline: 12.6700x; 12.6700x over previous
import dataclasses

import jax
import jax.numpy as jnp
from jax import lax
from jax.experimental import pallas as pl
from jax.experimental.pallas import tpu as pltpu
from jax.experimental.pallas import tpu_sc as plsc

NN = 10000
EE = 320000
HH = 128
KK = 8
NP = 10240
EP = 327680
NWORK = 32
PER_TILE = EP // NWORK
CH = 128
NCHUNK = PER_TILE // CH
ROWS_PER_TILE = NP // 16
_HI = jax.lax.Precision.HIGHEST

def _sc_mesh():
    return plsc.VectorSubcoreMesh(
        core_axis_name="c", subcore_axis_name="s", num_cores=2, num_subcores=16
    )


def _sc_params():
    cp = pltpu.CompilerParams()
    if "needs_layout_passes" in pltpu.CompilerParams.__dataclass_fields__:
        cp = dataclasses.replace(cp, needs_layout_passes=False)
    return cp


def _node_body(h_ref, W1_ref, b1_ref, W2_ref, b2_ref, A_ref, bc1_ref,
               B_ref, pi_ref, sig_ref, a_ref, c_ref, trow_ref, tcol_ref):
    hb = h_ref[...]
    t = jnp.dot(hb, W1_ref[...], precision=_HI) + b1_ref[...]
    t = t / (1.0 + jnp.exp(-t))
    sp = jnp.dot(t, W2_ref[...], precision=_HI) + b2_ref[...]

    p = sp[:, 0:8]
    m = jnp.max(p, axis=1, keepdims=True)
    e = jnp.exp(p - m)
    pi_ref[...] = e / jnp.sum(e, axis=1, keepdims=True)

    sg = sp[:, 8:16]
    sig_ref[...] = jnp.maximum(sg, 0.0) + jnp.log1p(jnp.exp(-jnp.abs(sg))) + 1e-5

    q = sp[:, 16:144]
    m = jnp.max(q, axis=1, keepdims=True)
    e = jnp.exp(q - m)
    r = lax.broadcasted_iota(jnp.int32, (128, 8), 0)
    k = lax.broadcasted_iota(jnp.int32, (128, 8), 1)
    G = ((r // 16) == k).astype(jnp.float32)
    rT = lax.broadcasted_iota(jnp.int32, (8, 128), 0)
    kT = lax.broadcasted_iota(jnp.int32, (8, 128), 1)
    GT = (rT == (kT // 16)).astype(jnp.float32)
    den = jnp.dot(jnp.dot(e, G, precision=_HI), GT, precision=_HI)
    a_ref[...] = e / den

    q = sp[:, 144:192]
    m = jnp.max(q, axis=1, keepdims=True)
    e = jnp.exp(q - m)
    r = lax.broadcasted_iota(jnp.int32, (48, 8), 0)
    k = lax.broadcasted_iota(jnp.int32, (48, 8), 1)
    G6 = ((r // 6) == k).astype(jnp.float32)
    rT = lax.broadcasted_iota(jnp.int32, (8, 48), 0)
    kT = lax.broadcasted_iota(jnp.int32, (8, 48), 1)
    G6T = (rT == (kT // 6)).astype(jnp.float32)
    den = jnp.dot(jnp.dot(e, G6, precision=_HI), G6T, precision=_HI)
    c_ref[...] = e / den

    trow_ref[...] = jnp.dot(hb, A_ref[...], precision=_HI) + bc1_ref[...]
    tcol_ref[...] = jnp.dot(hb, B_ref[...], precision=_HI)


def _node_call(hp, W1, b1r, W2, b2r, A, bc1r, B):
    nb = NP // 1024
    full = lambda s: pl.BlockSpec(s, lambda i: (0,) * len(s))
    blk = lambda w: pl.BlockSpec((1024, w), lambda i: (i, 0))
    return pl.pallas_call(
        _node_body,
        grid=(nb,),
        in_specs=[blk(128), full((128, 128)), full((1, 128)),
                  full((128, 192)), full((1, 192)), full((128, 128)),
                  full((1, 128)), full((128, 128))],
        out_specs=[blk(8), blk(8), blk(128), blk(48), blk(128), blk(128)],
        out_shape=[
            jax.ShapeDtypeStruct((NP, 8), jnp.float32),
            jax.ShapeDtypeStruct((NP, 8), jnp.float32),
            jax.ShapeDtypeStruct((NP, 128), jnp.float32),
            jax.ShapeDtypeStruct((NP, 48), jnp.float32),
            jax.ShapeDtypeStruct((NP, 128), jnp.float32),
            jax.ShapeDtypeStruct((NP, 128), jnp.float32),
        ],
    )(hp, W1, b1r, W2, b2r, A, bc1r, B)


def _sc_gather_body(trow, tcol, rowg, colg, wd, x0, x1, x2, z_out, d_out,
                    idr, idc, rbuf, cbuf, zbuf, dbuf, wdbuf,
                    xb0, xb1, xb2):
    cid = lax.axis_index("c")
    sid = lax.axis_index("s")
    wid = sid * 2 + cid
    pltpu.sync_copy(wd, wdbuf)
    pltpu.sync_copy(x0, xb0)
    pltpu.sync_copy(x1, xb1)
    pltpu.sync_copy(x2, xb2)
    wdv = [wdbuf[pl.ds(16 * j, 16)] for j in range(8)]

    @pl.loop(0, CH)
    def _zero(i):
        dbuf[i, :] = jnp.zeros((16,), jnp.float32)

    @pl.loop(0, NCHUNK)
    def _chunk(ci):
        base = wid * PER_TILE + ci * CH
        pltpu.sync_copy(rowg.at[pl.ds(base, CH)], idr)
        pltpu.sync_copy(colg.at[pl.ds(base, CH)], idc)
        pltpu.sync_copy(trow.at[idr], rbuf)
        pltpu.sync_copy(tcol.at[idc], cbuf)

        for g in range(8):
            rowi = idr[pl.ds(16 * g, 16)]
            coli = idc[pl.ds(16 * g, 16)]
            dx = plsc.load_gather(xb0, [rowi]) - plsc.load_gather(xb0, [coli])
            dy = plsc.load_gather(xb1, [rowi]) - plsc.load_gather(xb1, [coli])
            dz = plsc.load_gather(xb2, [rowi]) - plsc.load_gather(xb2, [coli])
            ii = lax.broadcasted_iota(jnp.int32, (16,), 0) + 16 * g
            plsc.store_scatter(dbuf, [ii, jnp.full((16,), 0, jnp.int32)], dx)
            plsc.store_scatter(dbuf, [ii, jnp.full((16,), 1, jnp.int32)], dy)
            plsc.store_scatter(dbuf, [ii, jnp.full((16,), 2, jnp.int32)], dz)

        @pl.loop(0, CH)
        def _edge(e):
            dvec = dbuf[e, :]
            dist = jnp.sum(dvec * dvec)
            for j in range(8):
                zbuf[e, pl.ds(16 * j, 16)] = (
                    rbuf[e, pl.ds(16 * j, 16)]
                    + cbuf[e, pl.ds(16 * j, 16)]
                    + dist * wdv[j]
                )

        pltpu.sync_copy(zbuf, z_out.at[pl.ds(base, CH)])
        pltpu.sync_copy(dbuf, d_out.at[pl.ds(base, CH)])


def _sc_gather_call(trow, tcol, rowg, colg, wd, x0, x1, x2):
    f = pl.kernel(
        _sc_gather_body,
        out_type=(jax.ShapeDtypeStruct((EP, 128), jnp.float32),
                  jax.ShapeDtypeStruct((EP, 16), jnp.float32)),
        mesh=_sc_mesh(),
        scratch_types=[
            pltpu.VMEM((CH,), jnp.int32),
            pltpu.VMEM((CH,), jnp.int32),
            pltpu.VMEM((CH, 128), jnp.float32),
            pltpu.VMEM((CH, 128), jnp.float32),
            pltpu.VMEM((CH, 128), jnp.float32),
            pltpu.VMEM((CH, 16), jnp.float32),
            pltpu.VMEM((128,), jnp.float32),
            pltpu.VMEM((NP,), jnp.float32),
            pltpu.VMEM((NP,), jnp.float32),
            pltpu.VMEM((NP,), jnp.float32),
        ],
        compiler_params=_sc_params(),
    )
    return f(trow, tcol, rowg, colg, wd, x0, x1, x2)


def _emlp_body(z_ref, d_ref, Wc2_ref, bc2_ref, v_ref):
    z = z_ref[...]
    s = z / (1.0 + jnp.exp(-z))
    w = jnp.dot(s, Wc2_ref[...], precision=_HI) + bc2_ref[...]
    r8 = lax.broadcasted_iota(jnp.int32, (8, 32), 0)
    j8 = lax.broadcasted_iota(jnp.int32, (8, 32), 1)
    R8 = ((j8 // 4) == r8).astype(jnp.float32)
    r16 = lax.broadcasted_iota(jnp.int32, (16, 32), 0)
    j16 = lax.broadcasted_iota(jnp.int32, (16, 32), 1)
    R16 = ((j16 % 4) == r16).astype(jnp.float32)
    v_ref[...] = (jnp.dot(w, R8, precision=_HI)
                  * jnp.dot(d_ref[...], R16, precision=_HI))


def _emlp_call(z, d, Wc2, bc2r):
    be = 2048
    nb = EP // be
    full = lambda s: pl.BlockSpec(s, lambda i: (0,) * len(s))
    blk = lambda w: pl.BlockSpec((be, w), lambda i: (i, 0))
    return pl.pallas_call(
        _emlp_body,
        grid=(nb,),
        in_specs=[blk(128), blk(16), full((128, 8)), full((1, 8))],
        out_specs=blk(32),
        out_shape=jax.ShapeDtypeStruct((EP, 32), jnp.float32),
    )(z, d, Wc2, bc2r)


def _sc_scatter_body(rows, vals4, out, idx, vbuf, acc):
    cid = lax.axis_index("c")
    sid = lax.axis_index("s")
    wid = sid * 2 + cid
    iota = lax.broadcasted_iota(jnp.int32, (16,), 0)

    for q in range(4):
        @pl.loop(0, (NP * 8) // 16)
        def _z(i):
            acc[pl.ds(16 * i, 16)] = jnp.zeros((16,), jnp.float32)

        half = q // 2
        lo = 8 * (q % 2)
        col16 = iota - lo
        msk = (iota >= lo) & (iota < lo + 8)

        @pl.loop(0, NCHUNK)
        def _chunk(ci):
            base = wid * PER_TILE + ci * CH
            pltpu.sync_copy(rows.at[pl.ds(base, CH)], idx)
            pltpu.sync_copy(vals4.at[pl.ds(wid * (PER_TILE // 4) + ci * (CH // 4),
                                           CH // 4)], vbuf)

            @pl.loop(0, CH // 16)
            def _grp(g):
                rowv = idx[pl.ds(16 * g, 16)]
                for el in range(16):
                    tgt = rowv[el] * 8 + col16
                    v = vbuf[4 * g + el // 4,
                             pl.ds(32 * (el % 4) + 16 * half, 16)]
                    plsc.addupdate_scatter(acc, [tgt], v, mask=msk)

        pltpu.sync_copy(acc, out.at[q, wid])


def _sc_scatter_call(rows, vals4):
    f = pl.kernel(
        _sc_scatter_body,
        out_type=jax.ShapeDtypeStruct((4, NWORK, NP * 8), jnp.float32),
        mesh=_sc_mesh(),
        scratch_types=[
            pltpu.VMEM((CH,), jnp.int32),
            pltpu.VMEM((CH // 4, 128), jnp.float32),
            pltpu.VMEM((NP * 8,), jnp.float32),
        ],
        compiler_params=_sc_params(),
    )
    return f(rows, vals4)


def _musum_body(acc_ref, o_ref):
    o_ref[...] = jnp.sum(acc_ref[...], axis=1)


def _musum_call(a4):
    nb = (NP // 16) // 64
    return pl.pallas_call(
        _musum_body,
        grid=(nb,),
        in_specs=[pl.BlockSpec((4, NWORK, 64, 128), lambda i: (0, 0, i, 0))],
        out_specs=pl.BlockSpec((4, 64, 128), lambda i: (0, i, 0)),
        out_shape=jax.ShapeDtypeStruct((4, NP // 16, 128), jnp.float32),
    )(a4)


def _mu_body(s32_ref, x16_ref, mu_ref):
    s32 = s32_ref[...]
    rP = lax.broadcasted_iota(jnp.int32, (32, 24), 0)
    cP = lax.broadcasted_iota(jnp.int32, (32, 24), 1)
    P = ((cP == 3 * (rP // 4) + (rP % 4)) & ((rP % 4) < 3)).astype(jnp.float32)
    rQ = lax.broadcasted_iota(jnp.int32, (16, 24), 0)
    cQ = lax.broadcasted_iota(jnp.int32, (16, 24), 1)
    Q = ((cQ % 3) == rQ).astype(jnp.float32)
    mu_ref[...] = (jnp.dot(s32, P, precision=_HI)
                   + jnp.dot(x16_ref[...], Q, precision=_HI))


def _mu_call(accs, x16):
    a4 = accs.reshape(4, NWORK, NP // 16, 128)
    ssum = _musum_call(a4)
    s32 = jnp.moveaxis(ssum.reshape(4, NP, 8), 0, 1).reshape(NP, 32)
    bn = 1024
    nb = NP // bn
    return pl.pallas_call(
        _mu_body,
        grid=(nb,),
        in_specs=[pl.BlockSpec((bn, 32), lambda i: (i, 0)),
                  pl.BlockSpec((bn, 16), lambda i: (i, 0))],
        out_specs=pl.BlockSpec((bn, 24), lambda i: (i, 0)),
        out_shape=jax.ShapeDtypeStruct((NP, 24), jnp.float32),
    )(s32, x16)


def kernel(h, x, edge_index, W1, b1, W2, b2, Wc1, bc1, Wc2, bc2):
    hp = jnp.pad(h, ((0, NP - NN), (0, 0)))
    x16 = jnp.zeros((NP, 16), jnp.float32).at[:NN, :3].set(x)
    row = edge_index[0]
    col = edge_index[1]
    rowg = jnp.pad(row, (0, EP - EE))
    colg = jnp.pad(col, (0, EP - EE))
    rows = jnp.pad(row, (0, EP - EE), constant_values=NN)
    A = Wc1[:HH]
    B = Wc1[HH:2 * HH]
    wd = Wc1[2 * HH]

    pi, sigma, aa, cc, trow, tcol = _node_call(
        hp, W1, b1[None], W2, b2[None], A, bc1[None], B)
    z, dv = _sc_gather_call(trow, tcol, rowg, colg, wd,
                            x16[:, 0], x16[:, 1], x16[:, 2])
    vals = _emlp_call(z, dv, Wc2, bc2[None])
    accs = _sc_scatter_call(rows, vals.reshape(EP // 4, 128))
    mu24 = _mu_call(accs, x16)

    return (pi[:NN],
            mu24[:NN].reshape(NN, KK, 3),
            sigma[:NN],
            aa[:NN].reshape(NN, KK, 16),
            cc[:NN].reshape(NN, KK, 6))


# speedup vs baseline: 16.0838x; 1.2694x over previous
import dataclasses

import jax
import jax.numpy as jnp
from jax import lax
from jax.experimental import pallas as pl
from jax.experimental.pallas import tpu as pltpu
from jax.experimental.pallas import tpu_sc as plsc

NN = 10000
EE = 320000
HH = 128
KK = 8
NP = 10240
EP = 327680
NWORK = 32
PER_TILE = EP // NWORK
CH = 80
NCHUNK = PER_TILE // CH
ROWS_PER_TILE = NP // 16
_HI = jax.lax.Precision.HIGHEST

def _sc_mesh():
    return plsc.VectorSubcoreMesh(
        core_axis_name="c", subcore_axis_name="s", num_cores=2, num_subcores=16
    )


def _sc_params():
    cp = pltpu.CompilerParams()
    if "needs_layout_passes" in pltpu.CompilerParams.__dataclass_fields__:
        cp = dataclasses.replace(cp, needs_layout_passes=False)
    return cp


def _node_body(h_ref, W1_ref, b1_ref, W2_ref, b2_ref, A_ref, bc1_ref,
               B_ref, pi_ref, sig_ref, a_ref, c_ref, trow_ref, tcol_ref):
    hb = h_ref[...]
    t = jnp.dot(hb, W1_ref[...], precision=_HI) + b1_ref[...]
    t = t / (1.0 + jnp.exp(-t))
    sp = jnp.dot(t, W2_ref[...], precision=_HI) + b2_ref[...]

    p = sp[:, 0:8]
    m = jnp.max(p, axis=1, keepdims=True)
    e = jnp.exp(p - m)
    pi_ref[...] = e / jnp.sum(e, axis=1, keepdims=True)

    sg = sp[:, 8:16]
    sig_ref[...] = jnp.maximum(sg, 0.0) + jnp.log1p(jnp.exp(-jnp.abs(sg))) + 1e-5

    q = sp[:, 16:144]
    m = jnp.max(q, axis=1, keepdims=True)
    e = jnp.exp(q - m)
    r = lax.broadcasted_iota(jnp.int32, (128, 8), 0)
    k = lax.broadcasted_iota(jnp.int32, (128, 8), 1)
    G = ((r // 16) == k).astype(jnp.float32)
    rT = lax.broadcasted_iota(jnp.int32, (8, 128), 0)
    kT = lax.broadcasted_iota(jnp.int32, (8, 128), 1)
    GT = (rT == (kT // 16)).astype(jnp.float32)
    den = jnp.dot(jnp.dot(e, G, precision=_HI), GT, precision=_HI)
    a_ref[...] = e / den

    q = sp[:, 144:192]
    m = jnp.max(q, axis=1, keepdims=True)
    e = jnp.exp(q - m)
    r = lax.broadcasted_iota(jnp.int32, (48, 8), 0)
    k = lax.broadcasted_iota(jnp.int32, (48, 8), 1)
    G6 = ((r // 6) == k).astype(jnp.float32)
    rT = lax.broadcasted_iota(jnp.int32, (8, 48), 0)
    kT = lax.broadcasted_iota(jnp.int32, (8, 48), 1)
    G6T = (rT == (kT // 6)).astype(jnp.float32)
    den = jnp.dot(jnp.dot(e, G6, precision=_HI), G6T, precision=_HI)
    c_ref[...] = e / den

    trow_ref[...] = jnp.dot(hb, A_ref[...], precision=_HI) + bc1_ref[...]
    tcol_ref[...] = jnp.dot(hb, B_ref[...], precision=_HI)


def _node_call(hp, W1, b1r, W2, b2r, A, bc1r, B):
    nb = NP // 1024
    full = lambda s: pl.BlockSpec(s, lambda i: (0,) * len(s))
    blk = lambda w: pl.BlockSpec((1024, w), lambda i: (i, 0))
    return pl.pallas_call(
        _node_body,
        grid=(nb,),
        in_specs=[blk(128), full((128, 128)), full((1, 128)),
                  full((128, 192)), full((1, 192)), full((128, 128)),
                  full((1, 128)), full((128, 128))],
        out_specs=[blk(8), blk(8), blk(128), blk(48), blk(128), blk(128)],
        out_shape=[
            jax.ShapeDtypeStruct((NP, 8), jnp.float32),
            jax.ShapeDtypeStruct((NP, 8), jnp.float32),
            jax.ShapeDtypeStruct((NP, 128), jnp.float32),
            jax.ShapeDtypeStruct((NP, 48), jnp.float32),
            jax.ShapeDtypeStruct((NP, 128), jnp.float32),
            jax.ShapeDtypeStruct((NP, 128), jnp.float32),
        ],
    )(hp, W1, b1r, W2, b2r, A, bc1r, B)


SUP = 16


def _sc_gather_body(trow, tcol, rowg, colg, wd, x0, x1, x2, z_out, d_out,
                    idr, idc, rbuf0, cbuf0, rbuf1, cbuf1, zbuf, dbuf, wdbuf,
                    xb0, xb1, xb2, sg0, sg1, so):
    cid = lax.axis_index("c")
    sid = lax.axis_index("s")
    wid = sid * 2 + cid
    pltpu.sync_copy(wd, wdbuf)
    pltpu.sync_copy(x0, xb0)
    pltpu.sync_copy(x1, xb1)
    pltpu.sync_copy(x2, xb2)
    wdv = [wdbuf[pl.ds(16 * j, 16)] for j in range(8)]

    @pl.loop(0, CH)
    def _zero(i):
        dbuf[i, :] = jnp.zeros((16,), jnp.float32)

    def _compute(rbuf, cbuf, base):
        for g in range(CH // 16):
            rowi = idr[pl.ds(base + 16 * g, 16)]
            coli = idc[pl.ds(base + 16 * g, 16)]
            dx = plsc.load_gather(xb0, [rowi]) - plsc.load_gather(xb0, [coli])
            dy = plsc.load_gather(xb1, [rowi]) - plsc.load_gather(xb1, [coli])
            dz = plsc.load_gather(xb2, [rowi]) - plsc.load_gather(xb2, [coli])
            ii = lax.broadcasted_iota(jnp.int32, (16,), 0) + 16 * g
            plsc.store_scatter(dbuf, [ii, jnp.full((16,), 0, jnp.int32)], dx)
            plsc.store_scatter(dbuf, [ii, jnp.full((16,), 1, jnp.int32)], dy)
            plsc.store_scatter(dbuf, [ii, jnp.full((16,), 2, jnp.int32)], dz)

        @pl.loop(0, CH)
        def _edge(e):
            dvec = dbuf[e, :]
            dist = jnp.sum(dvec * dvec)
            for j in range(8):
                zbuf[e, pl.ds(16 * j, 16)] = (
                    rbuf[e, pl.ds(16 * j, 16)]
                    + cbuf[e, pl.ds(16 * j, 16)]
                    + dist * wdv[j]
                )

    @pl.loop(0, NCHUNK, step=2)
    def _chunk(ci):
        @pl.when(lax.rem(ci, SUP) == 0)
        def _():
            pltpu.sync_copy(rowg.at[pl.ds(wid * PER_TILE + ci * CH, SUP * CH)],
                            idr)
            pltpu.sync_copy(colg.at[pl.ds(wid * PER_TILE + ci * CH, SUP * CH)],
                            idc)
        off0 = lax.rem(ci, SUP) * CH
        g0a = pltpu.async_copy(trow.at[idr.at[pl.ds(off0, CH)]], rbuf0, sg0)
        g0b = pltpu.async_copy(tcol.at[idc.at[pl.ds(off0, CH)]], cbuf0, sg0)
        g1a = pltpu.async_copy(trow.at[idr.at[pl.ds(off0 + CH, CH)]], rbuf1, sg1)
        g1b = pltpu.async_copy(tcol.at[idc.at[pl.ds(off0 + CH, CH)]], cbuf1, sg1)
        base = wid * PER_TILE + ci * CH
        g0a.wait()
        g0b.wait()
        _compute(rbuf0, cbuf0, off0)
        oa = pltpu.async_copy(zbuf, z_out.at[pl.ds(base, CH)], so)
        ob = pltpu.async_copy(dbuf, d_out.at[pl.ds(base, CH)], so)
        g1a.wait()
        g1b.wait()
        oa.wait()
        ob.wait()
        _compute(rbuf1, cbuf1, off0 + CH)
        oc = pltpu.async_copy(zbuf, z_out.at[pl.ds(base + CH, CH)], so)
        od = pltpu.async_copy(dbuf, d_out.at[pl.ds(base + CH, CH)], so)
        oc.wait()
        od.wait()


def _sc_gather_call(trow, tcol, rowg, colg, wd, x0, x1, x2):
    f = pl.kernel(
        _sc_gather_body,
        out_type=(jax.ShapeDtypeStruct((EP, 128), jnp.float32),
                  jax.ShapeDtypeStruct((EP, 16), jnp.float32)),
        mesh=_sc_mesh(),
        scratch_types=[
            pltpu.VMEM((SUP * CH,), jnp.int32),
            pltpu.VMEM((SUP * CH,), jnp.int32),
            pltpu.VMEM((CH, 128), jnp.float32),
            pltpu.VMEM((CH, 128), jnp.float32),
            pltpu.VMEM((CH, 128), jnp.float32),
            pltpu.VMEM((CH, 128), jnp.float32),
            pltpu.VMEM((CH, 128), jnp.float32),
            pltpu.VMEM((CH, 16), jnp.float32),
            pltpu.VMEM((128,), jnp.float32),
            pltpu.VMEM((NP,), jnp.float32),
            pltpu.VMEM((NP,), jnp.float32),
            pltpu.VMEM((NP,), jnp.float32),
            pltpu.SemaphoreType.DMA,
            pltpu.SemaphoreType.DMA,
            pltpu.SemaphoreType.DMA,
        ],
        compiler_params=_sc_params(),
    )
    return f(trow, tcol, rowg, colg, wd, x0, x1, x2)


def _emlp_body(z_ref, d_ref, Wc2_ref, bc2_ref, v_ref):
    z = z_ref[...]
    s = z / (1.0 + jnp.exp(-z))
    w = jnp.dot(s, Wc2_ref[...], precision=_HI) + bc2_ref[...]
    r8 = lax.broadcasted_iota(jnp.int32, (8, 32), 0)
    j8 = lax.broadcasted_iota(jnp.int32, (8, 32), 1)
    R8 = ((j8 // 4) == r8).astype(jnp.float32)
    r16 = lax.broadcasted_iota(jnp.int32, (16, 32), 0)
    j16 = lax.broadcasted_iota(jnp.int32, (16, 32), 1)
    R16 = ((j16 % 4) == r16).astype(jnp.float32)
    v_ref[...] = (jnp.dot(w, R8, precision=_HI)
                  * jnp.dot(d_ref[...], R16, precision=_HI))


def _emlp_call(z, d, Wc2, bc2r):
    be = 2048
    nb = EP // be
    full = lambda s: pl.BlockSpec(s, lambda i: (0,) * len(s))
    blk = lambda w: pl.BlockSpec((be, w), lambda i: (i, 0))
    return pl.pallas_call(
        _emlp_body,
        grid=(nb,),
        in_specs=[blk(128), blk(16), full((128, 8)), full((1, 8))],
        out_specs=blk(32),
        out_shape=jax.ShapeDtypeStruct((EP, 32), jnp.float32),
    )(z, d, Wc2, bc2r)


CHS = 320
NCHS = PER_TILE // CHS


def _sc_scatter_body(rows, vals4, out, idxa, vb0, vb1, acc, sv0, sv1):
    cid = lax.axis_index("c")
    sid = lax.axis_index("s")
    wid = sid * 2 + cid
    iota = lax.broadcasted_iota(jnp.int32, (16,), 0)
    pltpu.sync_copy(rows.at[pl.ds(wid * PER_TILE, PER_TILE)], idxa)

    def _scatter(vbuf, ci, col16, msk, half):
        @pl.loop(0, CHS // 16)
        def _grp(g):
            rowv = idxa[pl.ds(ci * CHS + 16 * g, 16)]
            for el in range(16):
                tgt = rowv[el] * 8 + col16
                v = vbuf[4 * g + el // 4,
                         pl.ds(32 * (el % 4) + 16 * half, 16)]
                plsc.addupdate_scatter(acc, [tgt], v, mask=msk)

    for q in range(4):
        @pl.loop(0, (NP * 8) // 64)
        def _z(i):
            for u in range(4):
                acc[pl.ds(64 * i + 16 * u, 16)] = jnp.zeros((16,), jnp.float32)

        half = q // 2
        lo = 8 * (q % 2)
        col16 = iota - lo
        msk = (iota >= lo) & (iota < lo + 8)

        @pl.loop(0, NCHS, step=2)
        def _chunk(ci):
            vbase = wid * (PER_TILE // 4) + ci * (CHS // 4)
            d0 = pltpu.async_copy(vals4.at[pl.ds(vbase, CHS // 4)], vb0, sv0)
            d1 = pltpu.async_copy(vals4.at[pl.ds(vbase + CHS // 4, CHS // 4)],
                                  vb1, sv1)
            d0.wait()
            _scatter(vb0, ci, col16, msk, half)
            d1.wait()
            _scatter(vb1, ci + 1, col16, msk, half)

        pltpu.sync_copy(acc, out.at[q, wid])


def _sc_scatter_call(rows, vals4):
    f = pl.kernel(
        _sc_scatter_body,
        out_type=jax.ShapeDtypeStruct((4, NWORK, NP * 8), jnp.float32),
        mesh=_sc_mesh(),
        scratch_types=[
            pltpu.VMEM((PER_TILE,), jnp.int32),
            pltpu.VMEM((CHS // 4, 128), jnp.float32),
            pltpu.VMEM((CHS // 4, 128), jnp.float32),
            pltpu.VMEM((NP * 8,), jnp.float32),
            pltpu.SemaphoreType.DMA,
            pltpu.SemaphoreType.DMA,
        ],
        compiler_params=_sc_params(),
    )
    return f(rows, vals4)


def _musum_body(acc_ref, o_ref):
    o_ref[...] = jnp.sum(acc_ref[...], axis=1)


def _musum_call(a4):
    nb = (NP // 16) // 64
    return pl.pallas_call(
        _musum_body,
        grid=(nb,),
        in_specs=[pl.BlockSpec((4, NWORK, 64, 128), lambda i: (0, 0, i, 0))],
        out_specs=pl.BlockSpec((4, 64, 128), lambda i: (0, i, 0)),
        out_shape=jax.ShapeDtypeStruct((4, NP // 16, 128), jnp.float32),
    )(a4)


def _mu_body(s32_ref, x16_ref, mu_ref):
    s32 = s32_ref[...]
    rP = lax.broadcasted_iota(jnp.int32, (32, 24), 0)
    cP = lax.broadcasted_iota(jnp.int32, (32, 24), 1)
    P = ((cP == 3 * (rP // 4) + (rP % 4)) & ((rP % 4) < 3)).astype(jnp.float32)
    rQ = lax.broadcasted_iota(jnp.int32, (16, 24), 0)
    cQ = lax.broadcasted_iota(jnp.int32, (16, 24), 1)
    Q = ((cQ % 3) == rQ).astype(jnp.float32)
    mu_ref[...] = (jnp.dot(s32, P, precision=_HI)
                   + jnp.dot(x16_ref[...], Q, precision=_HI))


def _mu_call(accs, x16):
    a4 = accs.reshape(4, NWORK, NP // 16, 128)
    ssum = _musum_call(a4)
    s32 = jnp.moveaxis(ssum.reshape(4, NP, 8), 0, 1).reshape(NP, 32)
    bn = 1024
    nb = NP // bn
    return pl.pallas_call(
        _mu_body,
        grid=(nb,),
        in_specs=[pl.BlockSpec((bn, 32), lambda i: (i, 0)),
                  pl.BlockSpec((bn, 16), lambda i: (i, 0))],
        out_specs=pl.BlockSpec((bn, 24), lambda i: (i, 0)),
        out_shape=jax.ShapeDtypeStruct((NP, 24), jnp.float32),
    )(s32, x16)


def kernel(h, x, edge_index, W1, b1, W2, b2, Wc1, bc1, Wc2, bc2):
    hp = jnp.pad(h, ((0, NP - NN), (0, 0)))
    x16 = jnp.zeros((NP, 16), jnp.float32).at[:NN, :3].set(x)
    row = edge_index[0]
    col = edge_index[1]
    rowg = jnp.pad(row, (0, EP - EE))
    colg = jnp.pad(col, (0, EP - EE))
    rows = jnp.pad(row, (0, EP - EE), constant_values=NN)
    A = Wc1[:HH]
    B = Wc1[HH:2 * HH]
    wd = Wc1[2 * HH]

    pi, sigma, aa, cc, trow, tcol = _node_call(
        hp, W1, b1[None], W2, b2[None], A, bc1[None], B)
    z, dv = _sc_gather_call(trow, tcol, rowg, colg, wd,
                            x16[:, 0], x16[:, 1], x16[:, 2])
    vals = _emlp_call(z, dv, Wc2, bc2[None])
    accs = _sc_scatter_call(rows, vals.reshape(EP // 4, 128))
    mu24 = _mu_call(accs, x16)

    return (pi[:NN],
            mu24[:NN].reshape(NN, KK, 3),
            sigma[:NN],
            aa[:NN].reshape(NN, KK, 16),
            cc[:NN].reshape(NN, KK, 6))


# speedup vs baseline: 19.3092x; 1.2005x over previous
import dataclasses

import jax
import jax.numpy as jnp
from jax import lax
from jax.experimental import pallas as pl
from jax.experimental.pallas import tpu as pltpu
from jax.experimental.pallas import tpu_sc as plsc

NN = 10000
EE = 320000
HH = 128
KK = 8
NP = 10240
EP = 327680
NWORK = 32
PER_TILE = EP // NWORK
CH = 80
NCHUNK = PER_TILE // CH
ROWS_PER_TILE = NP // 16
_HI = jax.lax.Precision.HIGHEST

def _sc_mesh():
    return plsc.VectorSubcoreMesh(
        core_axis_name="c", subcore_axis_name="s", num_cores=2, num_subcores=16
    )


def _sc_params():
    cp = pltpu.CompilerParams()
    if "needs_layout_passes" in pltpu.CompilerParams.__dataclass_fields__:
        cp = dataclasses.replace(cp, needs_layout_passes=False)
    return cp


def _node_body(h_ref, W1_ref, b1_ref, W2_ref, b2_ref, A_ref, bc1_ref,
               B_ref, pi_ref, sig_ref, a_ref, c_ref, trow_ref, tcol_ref):
    hb = h_ref[...]
    t = jnp.dot(hb, W1_ref[...], precision=_HI) + b1_ref[...]
    t = t / (1.0 + jnp.exp(-t))
    sp = jnp.dot(t, W2_ref[...], precision=_HI) + b2_ref[...]

    p = sp[:, 0:8]
    m = jnp.max(p, axis=1, keepdims=True)
    e = jnp.exp(p - m)
    pi_ref[...] = e / jnp.sum(e, axis=1, keepdims=True)

    sg = sp[:, 8:16]
    sig_ref[...] = jnp.maximum(sg, 0.0) + jnp.log1p(jnp.exp(-jnp.abs(sg))) + 1e-5

    q = sp[:, 16:144]
    m = jnp.max(q, axis=1, keepdims=True)
    e = jnp.exp(q - m)
    r = lax.broadcasted_iota(jnp.int32, (128, 8), 0)
    k = lax.broadcasted_iota(jnp.int32, (128, 8), 1)
    G = ((r // 16) == k).astype(jnp.float32)
    rT = lax.broadcasted_iota(jnp.int32, (8, 128), 0)
    kT = lax.broadcasted_iota(jnp.int32, (8, 128), 1)
    GT = (rT == (kT // 16)).astype(jnp.float32)
    den = jnp.dot(jnp.dot(e, G, precision=_HI), GT, precision=_HI)
    a_ref[...] = e / den

    q = sp[:, 144:192]
    m = jnp.max(q, axis=1, keepdims=True)
    e = jnp.exp(q - m)
    r = lax.broadcasted_iota(jnp.int32, (48, 8), 0)
    k = lax.broadcasted_iota(jnp.int32, (48, 8), 1)
    G6 = ((r // 6) == k).astype(jnp.float32)
    rT = lax.broadcasted_iota(jnp.int32, (8, 48), 0)
    kT = lax.broadcasted_iota(jnp.int32, (8, 48), 1)
    G6T = (rT == (kT // 6)).astype(jnp.float32)
    den = jnp.dot(jnp.dot(e, G6, precision=_HI), G6T, precision=_HI)
    c_ref[...] = e / den

    trow_ref[...] = jnp.dot(hb, A_ref[...], precision=_HI) + bc1_ref[...]
    tcol_ref[...] = jnp.dot(hb, B_ref[...], precision=_HI)


def _node_call(hp, W1, b1r, W2, b2r, A, bc1r, B):
    nb = NP // 1024
    full = lambda s: pl.BlockSpec(s, lambda i: (0,) * len(s))
    blk = lambda w: pl.BlockSpec((1024, w), lambda i: (i, 0))
    return pl.pallas_call(
        _node_body,
        grid=(nb,),
        in_specs=[blk(128), full((128, 128)), full((1, 128)),
                  full((128, 192)), full((1, 192)), full((128, 128)),
                  full((1, 128)), full((128, 128))],
        out_specs=[blk(8), blk(8), blk(128), blk(48), blk(128), blk(128)],
        out_shape=[
            jax.ShapeDtypeStruct((NP, 8), jnp.float32),
            jax.ShapeDtypeStruct((NP, 8), jnp.float32),
            jax.ShapeDtypeStruct((NP, 128), jnp.float32),
            jax.ShapeDtypeStruct((NP, 48), jnp.float32),
            jax.ShapeDtypeStruct((NP, 128), jnp.float32),
            jax.ShapeDtypeStruct((NP, 128), jnp.float32),
        ],
    )(hp, W1, b1r, W2, b2r, A, bc1r, B)


SUP = 16


def _sc_gather_body(trow, tcol, rowg, colg, wd, x0, x1, x2, z_out, d_out,
                    idr, idc, rbuf0, cbuf0, rbuf1, cbuf1, zbuf, dbuf, wdbuf,
                    xb0, xb1, xb2, sg0, sg1, so):
    cid = lax.axis_index("c")
    sid = lax.axis_index("s")
    wid = sid * 2 + cid
    pltpu.sync_copy(wd, wdbuf)
    pltpu.sync_copy(x0, xb0)
    pltpu.sync_copy(x1, xb1)
    pltpu.sync_copy(x2, xb2)
    wdv = [wdbuf[pl.ds(16 * j, 16)] for j in range(8)]

    @pl.loop(0, CH)
    def _zero(i):
        dbuf[i, :] = jnp.zeros((16,), jnp.float32)

    def _compute(rbuf, cbuf, base):
        for g in range(CH // 16):
            rowi = idr[pl.ds(base + 16 * g, 16)]
            coli = idc[pl.ds(base + 16 * g, 16)]
            dx = plsc.load_gather(xb0, [rowi]) - plsc.load_gather(xb0, [coli])
            dy = plsc.load_gather(xb1, [rowi]) - plsc.load_gather(xb1, [coli])
            dz = plsc.load_gather(xb2, [rowi]) - plsc.load_gather(xb2, [coli])
            ii = lax.broadcasted_iota(jnp.int32, (16,), 0) + 16 * g
            plsc.store_scatter(dbuf, [ii, jnp.full((16,), 0, jnp.int32)], dx)
            plsc.store_scatter(dbuf, [ii, jnp.full((16,), 1, jnp.int32)], dy)
            plsc.store_scatter(dbuf, [ii, jnp.full((16,), 2, jnp.int32)], dz)

        @pl.loop(0, CH)
        def _edge(e):
            dvec = dbuf[e, :]
            dist = jnp.sum(dvec * dvec)
            for j in range(8):
                zbuf[e, pl.ds(16 * j, 16)] = (
                    rbuf[e, pl.ds(16 * j, 16)]
                    + cbuf[e, pl.ds(16 * j, 16)]
                    + dist * wdv[j]
                )

    pltpu.sync_copy(rowg.at[pl.ds(wid * PER_TILE, PER_TILE)], idr)
    pltpu.sync_copy(colg.at[pl.ds(wid * PER_TILE, PER_TILE)], idc)

    def _issue(ci, rbuf, cbuf, sem):
        pltpu.async_copy(trow.at[idr.at[pl.ds(ci * CH, CH)]], rbuf, sem)
        pltpu.async_copy(tcol.at[idc.at[pl.ds(ci * CH, CH)]], cbuf, sem)

    def _drain(rbuf, cbuf, sem):
        pltpu.make_async_copy(trow.at[idr.at[pl.ds(0, CH)]], rbuf, sem).wait()
        pltpu.make_async_copy(tcol.at[idc.at[pl.ds(0, CH)]], cbuf, sem).wait()

    _issue(0, rbuf0, cbuf0, sg0)
    _issue(1, rbuf1, cbuf1, sg1)

    @pl.loop(0, NCHUNK, step=2)
    def _chunk(ci):
        base = wid * PER_TILE + ci * CH
        _drain(rbuf0, cbuf0, sg0)
        _compute(rbuf0, cbuf0, ci * CH)

        @pl.when(ci < NCHUNK - 2)
        def _():
            _issue(ci + 2, rbuf0, cbuf0, sg0)

        oa = pltpu.async_copy(zbuf, z_out.at[pl.ds(base, CH)], so)
        ob = pltpu.async_copy(dbuf, d_out.at[pl.ds(base, CH)], so)
        _drain(rbuf1, cbuf1, sg1)
        oa.wait()
        ob.wait()
        _compute(rbuf1, cbuf1, (ci + 1) * CH)

        @pl.when(ci < NCHUNK - 2)
        def _():
            _issue(ci + 3, rbuf1, cbuf1, sg1)

        oc = pltpu.async_copy(zbuf, z_out.at[pl.ds(base + CH, CH)], so)
        od = pltpu.async_copy(dbuf, d_out.at[pl.ds(base + CH, CH)], so)
        oc.wait()
        od.wait()


def _sc_gather_call(trow, tcol, rowg, colg, wd, x0, x1, x2):
    f = pl.kernel(
        _sc_gather_body,
        out_type=(jax.ShapeDtypeStruct((EP, 128), jnp.float32),
                  jax.ShapeDtypeStruct((EP, 16), jnp.float32)),
        mesh=_sc_mesh(),
        scratch_types=[
            pltpu.VMEM((PER_TILE,), jnp.int32),
            pltpu.VMEM((PER_TILE,), jnp.int32),
            pltpu.VMEM((CH, 128), jnp.float32),
            pltpu.VMEM((CH, 128), jnp.float32),
            pltpu.VMEM((CH, 128), jnp.float32),
            pltpu.VMEM((CH, 128), jnp.float32),
            pltpu.VMEM((CH, 128), jnp.float32),
            pltpu.VMEM((CH, 16), jnp.float32),
            pltpu.VMEM((128,), jnp.float32),
            pltpu.VMEM((NP,), jnp.float32),
            pltpu.VMEM((NP,), jnp.float32),
            pltpu.VMEM((NP,), jnp.float32),
            pltpu.SemaphoreType.DMA,
            pltpu.SemaphoreType.DMA,
            pltpu.SemaphoreType.DMA,
        ],
        compiler_params=_sc_params(),
    )
    return f(trow, tcol, rowg, colg, wd, x0, x1, x2)


def _emlp_body(z_ref, d_ref, Wc2_ref, bc2_ref, v_ref):
    z = z_ref[...]
    s = z / (1.0 + jnp.exp(-z))
    w = jnp.dot(s, Wc2_ref[...], precision=_HI) + bc2_ref[...]
    r8 = lax.broadcasted_iota(jnp.int32, (8, 32), 0)
    j8 = lax.broadcasted_iota(jnp.int32, (8, 32), 1)
    R8 = ((j8 // 4) == r8).astype(jnp.float32)
    r16 = lax.broadcasted_iota(jnp.int32, (16, 32), 0)
    j16 = lax.broadcasted_iota(jnp.int32, (16, 32), 1)
    R16 = ((j16 % 4) == r16).astype(jnp.float32)
    v_ref[...] = (jnp.dot(w, R8, precision=_HI)
                  * jnp.dot(d_ref[...], R16, precision=_HI))


def _emlp_call(z, d, Wc2, bc2r):
    be = 2048
    nb = EP // be
    full = lambda s: pl.BlockSpec(s, lambda i: (0,) * len(s))
    blk = lambda w: pl.BlockSpec((be, w), lambda i: (i, 0))
    return pl.pallas_call(
        _emlp_body,
        grid=(nb,),
        in_specs=[blk(128), blk(16), full((128, 8)), full((1, 8))],
        out_specs=blk(32),
        out_shape=jax.ShapeDtypeStruct((EP, 32), jnp.float32),
    )(z, d, Wc2, bc2r)


CHS = 320
NCHS = PER_TILE // CHS


def _sc_scatter_body(rows, vals4, out, idxa, vb0, vb1, acc, sv0, sv1):
    cid = lax.axis_index("c")
    sid = lax.axis_index("s")
    wid = sid * 2 + cid
    iota = lax.broadcasted_iota(jnp.int32, (16,), 0)
    pltpu.sync_copy(rows.at[pl.ds(wid * PER_TILE, PER_TILE)], idxa)

    def _scatter(vbuf, ci, col16, msk, half):
        @pl.loop(0, CHS // 16)
        def _grp(g):
            rowv = idxa[pl.ds(ci * CHS + 16 * g, 16)]
            for el in range(16):
                tgt = rowv[el] * 8 + col16
                v = vbuf[4 * g + el // 4,
                         pl.ds(32 * (el % 4) + 16 * half, 16)]
                plsc.addupdate_scatter(acc, [tgt], v, mask=msk)

    for q in range(4):
        @pl.loop(0, (NP * 8) // 64)
        def _z(i):
            for u in range(4):
                acc[pl.ds(64 * i + 16 * u, 16)] = jnp.zeros((16,), jnp.float32)

        half = q // 2
        lo = 8 * (q % 2)
        col16 = iota - lo
        msk = (iota >= lo) & (iota < lo + 8)

        @pl.loop(0, NCHS, step=2)
        def _chunk(ci):
            vbase = wid * (PER_TILE // 4) + ci * (CHS // 4)
            d0 = pltpu.async_copy(vals4.at[pl.ds(vbase, CHS // 4)], vb0, sv0)
            d1 = pltpu.async_copy(vals4.at[pl.ds(vbase + CHS // 4, CHS // 4)],
                                  vb1, sv1)
            d0.wait()
            _scatter(vb0, ci, col16, msk, half)
            d1.wait()
            _scatter(vb1, ci + 1, col16, msk, half)

        pltpu.sync_copy(acc, out.at[q, wid])


def _sc_scatter_call(rows, vals4):
    f = pl.kernel(
        _sc_scatter_body,
        out_type=jax.ShapeDtypeStruct((4, NWORK, NP * 8), jnp.float32),
        mesh=_sc_mesh(),
        scratch_types=[
            pltpu.VMEM((PER_TILE,), jnp.int32),
            pltpu.VMEM((CHS // 4, 128), jnp.float32),
            pltpu.VMEM((CHS // 4, 128), jnp.float32),
            pltpu.VMEM((NP * 8,), jnp.float32),
            pltpu.SemaphoreType.DMA,
            pltpu.SemaphoreType.DMA,
        ],
        compiler_params=_sc_params(),
    )
    return f(rows, vals4)


def _musum_body(acc_ref, o_ref):
    o_ref[...] = jnp.sum(acc_ref[...], axis=1)


def _musum_call(a4):
    nb = (NP // 16) // 64
    return pl.pallas_call(
        _musum_body,
        grid=(nb,),
        in_specs=[pl.BlockSpec((4, NWORK, 64, 128), lambda i: (0, 0, i, 0))],
        out_specs=pl.BlockSpec((4, 64, 128), lambda i: (0, i, 0)),
        out_shape=jax.ShapeDtypeStruct((4, NP // 16, 128), jnp.float32),
    )(a4)


def _mu_body(s32_ref, x16_ref, mu_ref):
    s32 = s32_ref[...]
    rP = lax.broadcasted_iota(jnp.int32, (32, 24), 0)
    cP = lax.broadcasted_iota(jnp.int32, (32, 24), 1)
    P = ((cP == 3 * (rP // 4) + (rP % 4)) & ((rP % 4) < 3)).astype(jnp.float32)
    rQ = lax.broadcasted_iota(jnp.int32, (16, 24), 0)
    cQ = lax.broadcasted_iota(jnp.int32, (16, 24), 1)
    Q = ((cQ % 3) == rQ).astype(jnp.float32)
    mu_ref[...] = (jnp.dot(s32, P, precision=_HI)
                   + jnp.dot(x16_ref[...], Q, precision=_HI))


def _mu_call(accs, x16):
    a4 = accs.reshape(4, NWORK, NP // 16, 128)
    ssum = _musum_call(a4)
    s32 = jnp.moveaxis(ssum.reshape(4, NP, 8), 0, 1).reshape(NP, 32)
    bn = 1024
    nb = NP // bn
    return pl.pallas_call(
        _mu_body,
        grid=(nb,),
        in_specs=[pl.BlockSpec((bn, 32), lambda i: (i, 0)),
                  pl.BlockSpec((bn, 16), lambda i: (i, 0))],
        out_specs=pl.BlockSpec((bn, 24), lambda i: (i, 0)),
        out_shape=jax.ShapeDtypeStruct((NP, 24), jnp.float32),
    )(s32, x16)


def kernel(h, x, edge_index, W1, b1, W2, b2, Wc1, bc1, Wc2, bc2):
    hp = jnp.pad(h, ((0, NP - NN), (0, 0)))
    x16 = jnp.zeros((NP, 16), jnp.float32).at[:NN, :3].set(x)
    row = edge_index[0]
    col = edge_index[1]
    rowg = jnp.pad(row, (0, EP - EE))
    colg = jnp.pad(col, (0, EP - EE))
    rows = jnp.pad(row, (0, EP - EE), constant_values=NN)
    A = Wc1[:HH]
    B = Wc1[HH:2 * HH]
    wd = Wc1[2 * HH]

    pi, sigma, aa, cc, trow, tcol = _node_call(
        hp, W1, b1[None], W2, b2[None], A, bc1[None], B)
    z, dv = _sc_gather_call(trow, tcol, rowg, colg, wd,
                            x16[:, 0], x16[:, 1], x16[:, 2])
    vals = _emlp_call(z, dv, Wc2, bc2[None])
    accs = _sc_scatter_call(rows, vals.reshape(EP // 4, 128))
    mu24 = _mu_call(accs, x16)

    return (pi[:NN],
            mu24[:NN].reshape(NN, KK, 3),
            sigma[:NN],
            aa[:NN].reshape(NN, KK, 16),
            cc[:NN].reshape(NN, KK, 6))
